# initial kernel scaffold (unmeasured)
import jax
import jax.numpy as jnp
from jax import lax
from jax.experimental import pallas as pl
from jax.experimental.pallas import tpu as pltpu

N_DEV = 16
N_TOK = 2048
D_MODEL = 512
D_FF = 1024
N_EXPERTS = 64
E_LOCAL = N_EXPERTS // N_DEV
CHUNK = N_TOK // N_DEV
N_HOPS = N_DEV - 1


def kernel(x, router_W, route_idx, expert_W, shared_W):
    def body(x_ref, rw_ref, idx_ref, ew_ref, sw_ref, out_ref,
             acc_ref, comm_ref,
             rs_send_sems, rs_recv_sems, ag_send_sems, ag_recv_sems):
        my = lax.axis_index("i")
        left = lax.rem(my + N_DEV - 1, N_DEV)
        right = lax.rem(my + 1, N_DEV)

        barrier = pltpu.get_barrier_semaphore()
        for nbr in (left, right):
            pl.semaphore_signal(barrier, inc=1, device_id=(nbr,),
                                device_id_type=pl.DeviceIdType.MESH)
        pl.semaphore_wait(barrier, 2)

        xv = x_ref[:, :]
        scores = jnp.dot(xv, rw_ref[:, :],
                         preferred_element_type=jnp.float32)
        smax = jnp.max(scores, axis=-1, keepdims=True)
        es = jnp.exp(scores - smax)
        probs = es / jnp.sum(es, axis=-1, keepdims=True)

        idx = idx_ref[:, :]
        col = lax.broadcasted_iota(jnp.int32, (N_TOK, N_EXPERTS), 1)
        total = jnp.zeros((N_TOK, D_FF), jnp.float32)
        for k in range(E_LOCAL):
            e = my * E_LOCAL + k
            p_e = jnp.sum(jnp.where(col == e, probs, 0.0),
                          axis=1, keepdims=True)
            s = jnp.where(idx == e, p_e, 0.0)
            total = total + jnp.dot(xv * s, ew_ref[k],
                                    preferred_element_type=jnp.float32)

        @pl.when(my == 0)
        def _():
            acc_ref[:, :] = total + jnp.dot(
                xv, sw_ref[:, :], preferred_element_type=jnp.float32)

        @pl.when(my != 0)
        def _():
            acc_ref[:, :] = total

        for h in range(N_HOPS):
            c_s = lax.rem(my + N_DEV - h, N_DEV)
            if h == 0:
                src = acc_ref.at[pl.ds(c_s * CHUNK, CHUNK), :]
            else:
                comm_ref[h - 1, :, :] = (
                    comm_ref[h - 1, :, :]
                    + acc_ref[pl.ds(c_s * CHUNK, CHUNK), :])
                src = comm_ref.at[h - 1]
            rdma = pltpu.make_async_remote_copy(
                src_ref=src,
                dst_ref=comm_ref.at[h],
                send_sem=rs_send_sems.at[h],
                recv_sem=rs_recv_sems.at[h],
                device_id=(right,),
                device_id_type=pl.DeviceIdType.MESH,
            )
            rdma.start()
            rdma.wait()

        r = lax.rem(my + 1, N_DEV)
        out_ref[pl.ds(r * CHUNK, CHUNK), :] = (
            comm_ref[N_HOPS - 1, :, :]
            + acc_ref[pl.ds(r * CHUNK, CHUNK), :])

        for g in range(N_HOPS):
            c = lax.rem(my + 1 - g + N_DEV, N_DEV)
            rdma = pltpu.make_async_remote_copy(
                src_ref=out_ref.at[pl.ds(c * CHUNK, CHUNK), :],
                dst_ref=out_ref.at[pl.ds(c * CHUNK, CHUNK), :],
                send_sem=ag_send_sems.at[g],
                recv_sem=ag_recv_sems.at[g],
                device_id=(right,),
                device_id_type=pl.DeviceIdType.MESH,
            )
            rdma.start()
            rdma.wait()

    return pl.pallas_call(
        body,
        out_shape=jax.ShapeDtypeStruct((N_TOK, D_FF), jnp.float32),
        in_specs=[pl.BlockSpec(memory_space=pltpu.VMEM)] * 5,
        out_specs=pl.BlockSpec(memory_space=pltpu.VMEM),
        scratch_shapes=[
            pltpu.VMEM((N_TOK, D_FF), jnp.float32),
            pltpu.VMEM((N_HOPS, CHUNK, D_FF), jnp.float32),
            pltpu.SemaphoreType.DMA((N_HOPS,)),
            pltpu.SemaphoreType.DMA((N_HOPS,)),
            pltpu.SemaphoreType.DMA((N_HOPS,)),
            pltpu.SemaphoreType.DMA((N_HOPS,)),
        ],
        compiler_params=pltpu.CompilerParams(collective_id=0),
    )(x, router_W, route_idx, expert_W, shared_W)


# baseline (device time: 258253 ns/iter reference)
import jax
import jax.numpy as jnp
from jax import lax
from jax.experimental import pallas as pl
from jax.experimental.pallas import tpu as pltpu

N_DEV = 16
N_TOK = 2048
D_MODEL = 512
D_FF = 1024
N_EXPERTS = 64
E_LOCAL = N_EXPERTS // N_DEV
CHUNK = N_TOK // N_DEV
N_HOPS = N_DEV - 1


def kernel(x, router_W, route_idx, expert_W, shared_W):
    def body(x_ref, rw_ref, idx_ref, ew_ref, sw_ref, out_ref,
             acc_ref, comm_ref,
             rs_send_sems, rs_recv_sems, ag_send_sems, ag_recv_sems):
        my = lax.axis_index("i")
        left = lax.rem(my + N_DEV - 1, N_DEV)
        right = lax.rem(my + 1, N_DEV)

        barrier = pltpu.get_barrier_semaphore()
        for nbr in (left, right):
            pl.semaphore_signal(barrier, inc=1, device_id=(nbr,),
                                device_id_type=pl.DeviceIdType.MESH)
        pl.semaphore_wait(barrier, 2)

        xv = x_ref[:, :]
        scores = jnp.dot(xv, rw_ref[:, :],
                         preferred_element_type=jnp.float32)
        smax = jnp.max(scores, axis=-1, keepdims=True)
        es = jnp.exp(scores - smax)
        probs = es / jnp.sum(es, axis=-1, keepdims=True)

        idx = idx_ref[:, :]
        col = lax.broadcasted_iota(jnp.int32, (N_TOK, N_EXPERTS), 1)
        total = jnp.zeros((N_TOK, D_FF), jnp.float32)
        for k in range(E_LOCAL):
            e = my * E_LOCAL + k
            p_e = jnp.sum(jnp.where(col == e, probs, 0.0),
                          axis=1, keepdims=True)
            s = jnp.where(idx == e, p_e, 0.0)
            total = total + jnp.dot(xv * s, ew_ref[k],
                                    preferred_element_type=jnp.float32)

        @pl.when(my == 0)
        def _():
            acc_ref[:, :] = total + jnp.dot(
                xv, sw_ref[:, :], preferred_element_type=jnp.float32)

        @pl.when(my != 0)
        def _():
            acc_ref[:, :] = total

        for h in range(N_HOPS):
            c_s = lax.rem(my + N_DEV - h, N_DEV)
            if h == 0:
                src = acc_ref.at[pl.ds(c_s * CHUNK, CHUNK), :]
            else:
                comm_ref[h - 1, :, :] = (
                    comm_ref[h - 1, :, :]
                    + acc_ref[pl.ds(c_s * CHUNK, CHUNK), :])
                src = comm_ref.at[h - 1]
            rdma = pltpu.make_async_remote_copy(
                src_ref=src,
                dst_ref=comm_ref.at[h],
                send_sem=rs_send_sems.at[h],
                recv_sem=rs_recv_sems.at[h],
                device_id=(right,),
                device_id_type=pl.DeviceIdType.MESH,
            )
            rdma.start()
            rdma.wait()

        r = lax.rem(my + 1, N_DEV)
        out_ref[pl.ds(r * CHUNK, CHUNK), :] = (
            comm_ref[N_HOPS - 1, :, :]
            + acc_ref[pl.ds(r * CHUNK, CHUNK), :])

        for g in range(N_HOPS):
            c = lax.rem(my + 1 - g + N_DEV, N_DEV)
            rdma = pltpu.make_async_remote_copy(
                src_ref=out_ref.at[pl.ds(c * CHUNK, CHUNK), :],
                dst_ref=out_ref.at[pl.ds(c * CHUNK, CHUNK), :],
                send_sem=ag_send_sems.at[g],
                recv_sem=ag_recv_sems.at[g],
                device_id=(right,),
                device_id_type=pl.DeviceIdType.MESH,
            )
            rdma.start()
            rdma.wait()

    return pl.pallas_call(
        body,
        out_shape=jax.ShapeDtypeStruct((N_TOK, D_FF), jnp.float32),
        in_specs=[pl.BlockSpec(memory_space=pltpu.VMEM)] * 5,
        out_specs=pl.BlockSpec(memory_space=pltpu.VMEM),
        scratch_shapes=[
            pltpu.VMEM((N_TOK, D_FF), jnp.float32),
            pltpu.VMEM((N_HOPS, CHUNK, D_FF), jnp.float32),
            pltpu.SemaphoreType.DMA((N_HOPS,)),
            pltpu.SemaphoreType.DMA((N_HOPS,)),
            pltpu.SemaphoreType.DMA((N_HOPS,)),
            pltpu.SemaphoreType.DMA((N_HOPS,)),
        ],
        compiler_params=pltpu.CompilerParams(
            collective_id=0,
            vmem_limit_bytes=100 * 1024 * 1024,
        ),
    )(x, router_W, route_idx, expert_W, shared_W)


# device time: 213428 ns/iter; 1.2100x vs baseline; 1.2100x over previous
import jax
import jax.numpy as jnp
from jax import lax
from jax.experimental import pallas as pl
from jax.experimental.pallas import tpu as pltpu

N_DEV = 16
N_TOK = 2048
D_MODEL = 512
D_FF = 1024
HALF = D_FF // 2
N_EXPERTS = 64
E_LOCAL = N_EXPERTS // N_DEV
CHUNK = N_TOK // N_DEV
N_HOPS = N_DEV - 1


def kernel(x, router_W, route_idx, expert_W, shared_W):
    def body(x_ref, rw_ref, idx_ref, ew_ref, sw_ref, out_ref,
             acc_ref, comm_r_ref, comm_l_ref,
             rs_r_send, rs_r_recv, rs_l_send, rs_l_recv,
             ag_r_send, ag_r_recv, ag_l_send, ag_l_recv):
        my = lax.axis_index("i")
        left = lax.rem(my + N_DEV - 1, N_DEV)
        right = lax.rem(my + 1, N_DEV)

        barrier = pltpu.get_barrier_semaphore()
        for nbr in (left, right):
            pl.semaphore_signal(barrier, inc=1, device_id=(nbr,),
                                device_id_type=pl.DeviceIdType.MESH)
        pl.semaphore_wait(barrier, 2)

        xv = x_ref[:, :]
        scores = jnp.dot(xv, rw_ref[:, :],
                         preferred_element_type=jnp.float32)
        smax = jnp.max(scores, axis=-1, keepdims=True)
        es = jnp.exp(scores - smax)
        probs = es / jnp.sum(es, axis=-1, keepdims=True)

        idx = idx_ref[:, :]
        col = lax.broadcasted_iota(jnp.int32, (N_TOK, N_EXPERTS), 1)
        total = jnp.zeros((N_TOK, D_FF), jnp.float32)
        for k in range(E_LOCAL):
            e = my * E_LOCAL + k
            p_e = jnp.sum(jnp.where(col == e, probs, 0.0),
                          axis=1, keepdims=True)
            s = jnp.where(idx == e, p_e, 0.0)
            total = total + jnp.dot(xv * s, ew_ref[k],
                                    preferred_element_type=jnp.float32)

        @pl.when(my == 0)
        def _():
            acc_ref[:, :] = total + jnp.dot(
                xv, sw_ref[:, :], preferred_element_type=jnp.float32)

        @pl.when(my != 0)
        def _():
            acc_ref[:, :] = total

        for h in range(N_HOPS):
            c_r = lax.rem(my + N_DEV - h, N_DEV)
            c_l = lax.rem(my + h, N_DEV)
            if h == 0:
                src_r = acc_ref.at[pl.ds(c_r * CHUNK, CHUNK), pl.ds(0, HALF)]
                src_l = acc_ref.at[pl.ds(c_l * CHUNK, CHUNK), pl.ds(HALF, HALF)]
            else:
                comm_r_ref[h - 1, :, :] = (
                    comm_r_ref[h - 1, :, :]
                    + acc_ref[pl.ds(c_r * CHUNK, CHUNK), pl.ds(0, HALF)])
                comm_l_ref[h - 1, :, :] = (
                    comm_l_ref[h - 1, :, :]
                    + acc_ref[pl.ds(c_l * CHUNK, CHUNK), pl.ds(HALF, HALF)])
                src_r = comm_r_ref.at[h - 1]
                src_l = comm_l_ref.at[h - 1]
            rdma_r = pltpu.make_async_remote_copy(
                src_ref=src_r,
                dst_ref=comm_r_ref.at[h],
                send_sem=rs_r_send.at[h],
                recv_sem=rs_r_recv.at[h],
                device_id=(right,),
                device_id_type=pl.DeviceIdType.MESH,
            )
            rdma_l = pltpu.make_async_remote_copy(
                src_ref=src_l,
                dst_ref=comm_l_ref.at[h],
                send_sem=rs_l_send.at[h],
                recv_sem=rs_l_recv.at[h],
                device_id=(left,),
                device_id_type=pl.DeviceIdType.MESH,
            )
            rdma_r.start()
            rdma_l.start()
            rdma_r.wait()
            rdma_l.wait()

        r = lax.rem(my + 1, N_DEV)
        l = lax.rem(my + N_DEV - 1, N_DEV)
        out_ref[pl.ds(r * CHUNK, CHUNK), pl.ds(0, HALF)] = (
            comm_r_ref[N_HOPS - 1, :, :]
            + acc_ref[pl.ds(r * CHUNK, CHUNK), pl.ds(0, HALF)])
        out_ref[pl.ds(l * CHUNK, CHUNK), pl.ds(HALF, HALF)] = (
            comm_l_ref[N_HOPS - 1, :, :]
            + acc_ref[pl.ds(l * CHUNK, CHUNK), pl.ds(HALF, HALF)])

        for g in range(N_HOPS):
            c_r = lax.rem(my + 1 - g + N_DEV, N_DEV)
            c_l = lax.rem(my - 1 + g + N_DEV, N_DEV)
            rdma_r = pltpu.make_async_remote_copy(
                src_ref=out_ref.at[pl.ds(c_r * CHUNK, CHUNK), pl.ds(0, HALF)],
                dst_ref=out_ref.at[pl.ds(c_r * CHUNK, CHUNK), pl.ds(0, HALF)],
                send_sem=ag_r_send.at[g],
                recv_sem=ag_r_recv.at[g],
                device_id=(right,),
                device_id_type=pl.DeviceIdType.MESH,
            )
            rdma_l = pltpu.make_async_remote_copy(
                src_ref=out_ref.at[pl.ds(c_l * CHUNK, CHUNK), pl.ds(HALF, HALF)],
                dst_ref=out_ref.at[pl.ds(c_l * CHUNK, CHUNK), pl.ds(HALF, HALF)],
                send_sem=ag_l_send.at[g],
                recv_sem=ag_l_recv.at[g],
                device_id=(left,),
                device_id_type=pl.DeviceIdType.MESH,
            )
            rdma_r.start()
            rdma_l.start()
            rdma_r.wait()
            rdma_l.wait()

    return pl.pallas_call(
        body,
        out_shape=jax.ShapeDtypeStruct((N_TOK, D_FF), jnp.float32),
        in_specs=[pl.BlockSpec(memory_space=pltpu.VMEM)] * 5,
        out_specs=pl.BlockSpec(memory_space=pltpu.VMEM),
        scratch_shapes=[
            pltpu.VMEM((N_TOK, D_FF), jnp.float32),
            pltpu.VMEM((N_HOPS, CHUNK, HALF), jnp.float32),
            pltpu.VMEM((N_HOPS, CHUNK, HALF), jnp.float32),
            pltpu.SemaphoreType.DMA((N_HOPS,)),
            pltpu.SemaphoreType.DMA((N_HOPS,)),
            pltpu.SemaphoreType.DMA((N_HOPS,)),
            pltpu.SemaphoreType.DMA((N_HOPS,)),
            pltpu.SemaphoreType.DMA((N_HOPS,)),
            pltpu.SemaphoreType.DMA((N_HOPS,)),
            pltpu.SemaphoreType.DMA((N_HOPS,)),
            pltpu.SemaphoreType.DMA((N_HOPS,)),
        ],
        compiler_params=pltpu.CompilerParams(
            collective_id=0,
            vmem_limit_bytes=100 * 1024 * 1024,
        ),
    )(x, router_W, route_idx, expert_W, shared_W)


# device time: 206514 ns/iter; 1.2505x vs baseline; 1.0335x over previous
import jax
import jax.numpy as jnp
from jax import lax
from jax.experimental import pallas as pl
from jax.experimental.pallas import tpu as pltpu

N_DEV = 16
N_TOK = 2048
D_MODEL = 512
D_FF = 1024
HALF = D_FF // 2
N_EXPERTS = 64
E_LOCAL = N_EXPERTS // N_DEV
CHUNK = N_TOK // N_DEV
N_HOPS = N_DEV - 1


def kernel(x, router_W, route_idx, expert_W, shared_W):
    def body(x_ref, rw_ref, idx_ref, ew_ref, sw_ref, out_ref,
             acc_ref, probs_ref, comm_r_ref, comm_l_ref,
             rs_r_send, rs_r_recv, rs_l_send, rs_l_recv,
             ag_r_send, ag_r_recv, ag_l_send, ag_l_recv):
        my = lax.axis_index("i")
        left = lax.rem(my + N_DEV - 1, N_DEV)
        right = lax.rem(my + 1, N_DEV)

        barrier = pltpu.get_barrier_semaphore()
        for nbr in (left, right):
            pl.semaphore_signal(barrier, inc=1, device_id=(nbr,),
                                device_id_type=pl.DeviceIdType.MESH)
        pl.semaphore_wait(barrier, 2)

        scores = jnp.dot(x_ref[:, :], rw_ref[:, :],
                         preferred_element_type=jnp.float32)
        smax = jnp.max(scores, axis=-1, keepdims=True)
        es = jnp.exp(scores - smax)
        probs_ref[:, :] = es / jnp.sum(es, axis=-1, keepdims=True)

        col64 = lax.broadcasted_iota(jnp.int32, (CHUNK, N_EXPERTS), 1)

        def compute_half_chunk(c, col_lo):
            rows = pl.ds(c * CHUNK, CHUNK)
            cols = pl.ds(col_lo, HALF)
            xc = x_ref[rows, :]
            idxc = idx_ref[rows, :]
            pc = probs_ref[rows, :]
            t = jnp.zeros((CHUNK, HALF), jnp.float32)
            for k in range(E_LOCAL):
                e = my * E_LOCAL + k
                p_e = jnp.sum(jnp.where(col64 == e, pc, 0.0),
                              axis=1, keepdims=True)
                s = jnp.where(idxc == e, p_e, 0.0)
                t = t + jnp.dot(xc * s, ew_ref[k, :, cols],
                                preferred_element_type=jnp.float32)
            acc_ref[rows, cols] = t

            @pl.when(my == 0)
            def _():
                acc_ref[rows, cols] = acc_ref[rows, cols] + jnp.dot(
                    xc, sw_ref[:, cols], preferred_element_type=jnp.float32)

        for h in range(N_HOPS):
            c_r = lax.rem(my + N_DEV - h, N_DEV)
            c_l = lax.rem(my + h, N_DEV)
            if h == 0:
                compute_half_chunk(c_r, 0)
                compute_half_chunk(c_l, HALF)
                src_r = acc_ref.at[pl.ds(c_r * CHUNK, CHUNK), pl.ds(0, HALF)]
                src_l = acc_ref.at[pl.ds(c_l * CHUNK, CHUNK), pl.ds(HALF, HALF)]
            else:
                comm_r_ref[h - 1, :, :] = (
                    comm_r_ref[h - 1, :, :]
                    + acc_ref[pl.ds(c_r * CHUNK, CHUNK), pl.ds(0, HALF)])
                comm_l_ref[h - 1, :, :] = (
                    comm_l_ref[h - 1, :, :]
                    + acc_ref[pl.ds(c_l * CHUNK, CHUNK), pl.ds(HALF, HALF)])
                src_r = comm_r_ref.at[h - 1]
                src_l = comm_l_ref.at[h - 1]
            rdma_r = pltpu.make_async_remote_copy(
                src_ref=src_r,
                dst_ref=comm_r_ref.at[h],
                send_sem=rs_r_send.at[h],
                recv_sem=rs_r_recv.at[h],
                device_id=(right,),
                device_id_type=pl.DeviceIdType.MESH,
            )
            rdma_l = pltpu.make_async_remote_copy(
                src_ref=src_l,
                dst_ref=comm_l_ref.at[h],
                send_sem=rs_l_send.at[h],
                recv_sem=rs_l_recv.at[h],
                device_id=(left,),
                device_id_type=pl.DeviceIdType.MESH,
            )
            rdma_r.start()
            rdma_l.start()
            compute_half_chunk(lax.rem(my + N_DEV - h - 1, N_DEV), 0)
            compute_half_chunk(lax.rem(my + h + 1, N_DEV), HALF)
            rdma_r.wait()
            rdma_l.wait()

        r = lax.rem(my + 1, N_DEV)
        l = lax.rem(my + N_DEV - 1, N_DEV)
        out_ref[pl.ds(r * CHUNK, CHUNK), pl.ds(0, HALF)] = (
            comm_r_ref[N_HOPS - 1, :, :]
            + acc_ref[pl.ds(r * CHUNK, CHUNK), pl.ds(0, HALF)])
        out_ref[pl.ds(l * CHUNK, CHUNK), pl.ds(HALF, HALF)] = (
            comm_l_ref[N_HOPS - 1, :, :]
            + acc_ref[pl.ds(l * CHUNK, CHUNK), pl.ds(HALF, HALF)])

        for g in range(N_HOPS):
            c_r = lax.rem(my + 1 - g + N_DEV, N_DEV)
            c_l = lax.rem(my - 1 + g + N_DEV, N_DEV)
            rdma_r = pltpu.make_async_remote_copy(
                src_ref=out_ref.at[pl.ds(c_r * CHUNK, CHUNK), pl.ds(0, HALF)],
                dst_ref=out_ref.at[pl.ds(c_r * CHUNK, CHUNK), pl.ds(0, HALF)],
                send_sem=ag_r_send.at[g],
                recv_sem=ag_r_recv.at[g],
                device_id=(right,),
                device_id_type=pl.DeviceIdType.MESH,
            )
            rdma_l = pltpu.make_async_remote_copy(
                src_ref=out_ref.at[pl.ds(c_l * CHUNK, CHUNK), pl.ds(HALF, HALF)],
                dst_ref=out_ref.at[pl.ds(c_l * CHUNK, CHUNK), pl.ds(HALF, HALF)],
                send_sem=ag_l_send.at[g],
                recv_sem=ag_l_recv.at[g],
                device_id=(left,),
                device_id_type=pl.DeviceIdType.MESH,
            )
            rdma_r.start()
            rdma_l.start()
            rdma_r.wait()
            rdma_l.wait()

    return pl.pallas_call(
        body,
        out_shape=jax.ShapeDtypeStruct((N_TOK, D_FF), jnp.float32),
        in_specs=[pl.BlockSpec(memory_space=pltpu.VMEM)] * 5,
        out_specs=pl.BlockSpec(memory_space=pltpu.VMEM),
        scratch_shapes=[
            pltpu.VMEM((N_TOK, D_FF), jnp.float32),
            pltpu.VMEM((N_TOK, N_EXPERTS), jnp.float32),
            pltpu.VMEM((N_HOPS, CHUNK, HALF), jnp.float32),
            pltpu.VMEM((N_HOPS, CHUNK, HALF), jnp.float32),
            pltpu.SemaphoreType.DMA((N_HOPS,)),
            pltpu.SemaphoreType.DMA((N_HOPS,)),
            pltpu.SemaphoreType.DMA((N_HOPS,)),
            pltpu.SemaphoreType.DMA((N_HOPS,)),
            pltpu.SemaphoreType.DMA((N_HOPS,)),
            pltpu.SemaphoreType.DMA((N_HOPS,)),
            pltpu.SemaphoreType.DMA((N_HOPS,)),
            pltpu.SemaphoreType.DMA((N_HOPS,)),
        ],
        compiler_params=pltpu.CompilerParams(
            collective_id=0,
            vmem_limit_bytes=100 * 1024 * 1024,
        ),
    )(x, router_W, route_idx, expert_W, shared_W)


# device time: 206374 ns/iter; 1.2514x vs baseline; 1.0007x over previous
import jax
import jax.numpy as jnp
from jax import lax
from jax.experimental import pallas as pl
from jax.experimental.pallas import tpu as pltpu

N_DEV = 16
N_TOK = 2048
D_MODEL = 512
D_FF = 1024
HALF = D_FF // 2
N_EXPERTS = 64
E_LOCAL = N_EXPERTS // N_DEV
CHUNK = N_TOK // N_DEV
N_HOPS = N_DEV - 1


def kernel(x, router_W, route_idx, expert_W, shared_W):
    def body(x_ref, rw_ref, idx_ref, ew_ref, sw_ref, out_ref,
             acc_ref, probs_ref, comm_r_ref, comm_l_ref,
             rs_r_send, rs_r_recv, rs_l_send, rs_l_recv,
             ag_r_send, ag_r_recv, ag_l_send, ag_l_recv):
        my = lax.axis_index("i")
        left = lax.rem(my + N_DEV - 1, N_DEV)
        right = lax.rem(my + 1, N_DEV)

        barrier = pltpu.get_barrier_semaphore()
        for nbr in (left, right):
            pl.semaphore_signal(barrier, inc=1, device_id=(nbr,),
                                device_id_type=pl.DeviceIdType.MESH)
        pl.semaphore_wait(barrier, 2)

        scores = jnp.dot(x_ref[:, :], rw_ref[:, :],
                         preferred_element_type=jnp.float32)
        smax = jnp.max(scores, axis=-1, keepdims=True)
        es = jnp.exp(scores - smax)
        probs_ref[:, :] = es / jnp.sum(es, axis=-1, keepdims=True)

        col64 = lax.broadcasted_iota(jnp.int32, (CHUNK, N_EXPERTS), 1)

        def compute_half_chunk(c, col_lo):
            rows = pl.ds(c * CHUNK, CHUNK)
            cols = pl.ds(col_lo, HALF)
            xc = x_ref[rows, :]
            idxc = idx_ref[rows, :]
            pc = probs_ref[rows, :]
            xs = []
            for k in range(E_LOCAL):
                e = my * E_LOCAL + k
                p_e = jnp.sum(jnp.where(col64 == e, pc, 0.0),
                              axis=1, keepdims=True)
                s = jnp.where(idxc == e, p_e, 0.0)
                xs.append(xc * s)
            xs_cat = jnp.concatenate(xs, axis=1)
            w_cat = ew_ref[:, :, cols].reshape(E_LOCAL * D_MODEL, HALF)
            acc_ref[rows, cols] = jnp.dot(
                xs_cat, w_cat, preferred_element_type=jnp.float32)

            @pl.when(my == 0)
            def _():
                acc_ref[rows, cols] = acc_ref[rows, cols] + jnp.dot(
                    xc, sw_ref[:, cols], preferred_element_type=jnp.float32)

        for h in range(N_HOPS):
            c_r = lax.rem(my + N_DEV - h, N_DEV)
            c_l = lax.rem(my + h, N_DEV)
            if h == 0:
                compute_half_chunk(c_r, 0)
                compute_half_chunk(c_l, HALF)
                src_r = acc_ref.at[pl.ds(c_r * CHUNK, CHUNK), pl.ds(0, HALF)]
                src_l = acc_ref.at[pl.ds(c_l * CHUNK, CHUNK), pl.ds(HALF, HALF)]
            else:
                comm_r_ref[h - 1, :, :] = (
                    comm_r_ref[h - 1, :, :]
                    + acc_ref[pl.ds(c_r * CHUNK, CHUNK), pl.ds(0, HALF)])
                comm_l_ref[h - 1, :, :] = (
                    comm_l_ref[h - 1, :, :]
                    + acc_ref[pl.ds(c_l * CHUNK, CHUNK), pl.ds(HALF, HALF)])
                src_r = comm_r_ref.at[h - 1]
                src_l = comm_l_ref.at[h - 1]
            rdma_r = pltpu.make_async_remote_copy(
                src_ref=src_r,
                dst_ref=comm_r_ref.at[h],
                send_sem=rs_r_send.at[h],
                recv_sem=rs_r_recv.at[h],
                device_id=(right,),
                device_id_type=pl.DeviceIdType.MESH,
            )
            rdma_l = pltpu.make_async_remote_copy(
                src_ref=src_l,
                dst_ref=comm_l_ref.at[h],
                send_sem=rs_l_send.at[h],
                recv_sem=rs_l_recv.at[h],
                device_id=(left,),
                device_id_type=pl.DeviceIdType.MESH,
            )
            rdma_r.start()
            rdma_l.start()
            compute_half_chunk(lax.rem(my + N_DEV - h - 1, N_DEV), 0)
            compute_half_chunk(lax.rem(my + h + 1, N_DEV), HALF)
            rdma_r.wait()
            rdma_l.wait()

        r = lax.rem(my + 1, N_DEV)
        l = lax.rem(my + N_DEV - 1, N_DEV)
        out_ref[pl.ds(r * CHUNK, CHUNK), pl.ds(0, HALF)] = (
            comm_r_ref[N_HOPS - 1, :, :]
            + acc_ref[pl.ds(r * CHUNK, CHUNK), pl.ds(0, HALF)])
        out_ref[pl.ds(l * CHUNK, CHUNK), pl.ds(HALF, HALF)] = (
            comm_l_ref[N_HOPS - 1, :, :]
            + acc_ref[pl.ds(l * CHUNK, CHUNK), pl.ds(HALF, HALF)])

        for g in range(N_HOPS):
            c_r = lax.rem(my + 1 - g + N_DEV, N_DEV)
            c_l = lax.rem(my - 1 + g + N_DEV, N_DEV)
            rdma_r = pltpu.make_async_remote_copy(
                src_ref=out_ref.at[pl.ds(c_r * CHUNK, CHUNK), pl.ds(0, HALF)],
                dst_ref=out_ref.at[pl.ds(c_r * CHUNK, CHUNK), pl.ds(0, HALF)],
                send_sem=ag_r_send.at[g],
                recv_sem=ag_r_recv.at[g],
                device_id=(right,),
                device_id_type=pl.DeviceIdType.MESH,
            )
            rdma_l = pltpu.make_async_remote_copy(
                src_ref=out_ref.at[pl.ds(c_l * CHUNK, CHUNK), pl.ds(HALF, HALF)],
                dst_ref=out_ref.at[pl.ds(c_l * CHUNK, CHUNK), pl.ds(HALF, HALF)],
                send_sem=ag_l_send.at[g],
                recv_sem=ag_l_recv.at[g],
                device_id=(left,),
                device_id_type=pl.DeviceIdType.MESH,
            )
            rdma_r.start()
            rdma_l.start()
            rdma_r.wait()
            rdma_l.wait()

    return pl.pallas_call(
        body,
        out_shape=jax.ShapeDtypeStruct((N_TOK, D_FF), jnp.float32),
        in_specs=[pl.BlockSpec(memory_space=pltpu.VMEM)] * 5,
        out_specs=pl.BlockSpec(memory_space=pltpu.VMEM),
        scratch_shapes=[
            pltpu.VMEM((N_TOK, D_FF), jnp.float32),
            pltpu.VMEM((N_TOK, N_EXPERTS), jnp.float32),
            pltpu.VMEM((N_HOPS, CHUNK, HALF), jnp.float32),
            pltpu.VMEM((N_HOPS, CHUNK, HALF), jnp.float32),
            pltpu.SemaphoreType.DMA((N_HOPS,)),
            pltpu.SemaphoreType.DMA((N_HOPS,)),
            pltpu.SemaphoreType.DMA((N_HOPS,)),
            pltpu.SemaphoreType.DMA((N_HOPS,)),
            pltpu.SemaphoreType.DMA((N_HOPS,)),
            pltpu.SemaphoreType.DMA((N_HOPS,)),
            pltpu.SemaphoreType.DMA((N_HOPS,)),
            pltpu.SemaphoreType.DMA((N_HOPS,)),
        ],
        compiler_params=pltpu.CompilerParams(
            collective_id=0,
            vmem_limit_bytes=100 * 1024 * 1024,
        ),
    )(x, router_W, route_idx, expert_W, shared_W)


# device time: 123432 ns/iter; 2.0923x vs baseline; 1.6720x over previous
import jax
import jax.numpy as jnp
from jax import lax
from jax.experimental import pallas as pl
from jax.experimental.pallas import tpu as pltpu

N_DEV = 16
N_TOK = 2048
D_MODEL = 512
D_FF = 1024
HALF = D_FF // 2
N_EXPERTS = 64
E_LOCAL = N_EXPERTS // N_DEV
CHUNK = N_TOK // N_DEV
N_HOPS = N_DEV - 1


def kernel(x, router_W, route_idx, expert_W, shared_W):
    def body(x_ref, rw_ref, idx_ref, ew_ref, sw_ref, out_ref,
             acc_ref, probs_ref, agbuf_ref,
             slot_lor_ref, slot_lol_ref, slot_hil_ref, slot_hir_ref,
             rs_lor_send, rs_lor_recv, rs_lol_send, rs_lol_recv,
             rs_hil_send, rs_hil_recv, rs_hir_send, rs_hir_recv,
             ag_rhr_send, ag_rhr_recv, ag_lhl_send, ag_lhl_recv,
             ag_rhl_send, ag_rhl_recv, ag_lhr_send, ag_lhr_recv):
        my = lax.axis_index("i")
        left = lax.rem(my + N_DEV - 1, N_DEV)
        right = lax.rem(my + 1, N_DEV)

        barrier = pltpu.get_barrier_semaphore()
        for nbr in (left, right):
            pl.semaphore_signal(barrier, inc=1, device_id=(nbr,),
                                device_id_type=pl.DeviceIdType.MESH)
        pl.semaphore_wait(barrier, 2)

        scores = jnp.dot(x_ref[:, :], rw_ref[:, :],
                         preferred_element_type=jnp.float32)
        smax = jnp.max(scores, axis=-1, keepdims=True)
        es = jnp.exp(scores - smax)
        probs_ref[:, :] = es / jnp.sum(es, axis=-1, keepdims=True)

        col64 = lax.broadcasted_iota(jnp.int32, (CHUNK, N_EXPERTS), 1)

        def compute_half_chunk(c, col_lo):
            rows = pl.ds(c * CHUNK, CHUNK)
            cols = pl.ds(col_lo, HALF)
            xc = x_ref[rows, :]
            idxc = idx_ref[rows, :]
            pc = probs_ref[rows, :]
            xs = []
            for k in range(E_LOCAL):
                e = my * E_LOCAL + k
                p_e = jnp.sum(jnp.where(col64 == e, pc, 0.0),
                              axis=1, keepdims=True)
                s = jnp.where(idxc == e, p_e, 0.0)
                xs.append(xc * s)
            xs_cat = jnp.concatenate(xs, axis=1)
            w_cat = ew_ref[:, :, cols].reshape(E_LOCAL * D_MODEL, HALF)
            acc_ref[rows, cols] = jnp.dot(
                xs_cat, w_cat,
                preferred_element_type=jnp.float32).astype(jnp.bfloat16)

            @pl.when(my == 0)
            def _():
                acc_ref[rows, cols] = (
                    acc_ref[rows, cols].astype(jnp.float32) + jnp.dot(
                        xc, sw_ref[:, cols],
                        preferred_element_type=jnp.float32)
                ).astype(jnp.bfloat16)

        N_MIM = N_DEV // 2

        def lo_slice(c):
            return (pl.ds(c * CHUNK, CHUNK), pl.ds(0, HALF))

        def hi_slice(c):
            return (pl.ds(c * CHUNK, CHUNK), pl.ds(HALF, HALF))

        def rs_copy(src, dst_slots, ss, rs_, t, dev):
            return pltpu.make_async_remote_copy(
                src_ref=src,
                dst_ref=dst_slots.at[t],
                send_sem=ss.at[t],
                recv_sem=rs_.at[t],
                device_id=(dev,),
                device_id_type=pl.DeviceIdType.MESH,
            )

        compute_half_chunk(lax.rem(my + 9, N_DEV), 0)
        compute_half_chunk(lax.rem(my - 6 + N_DEV, N_DEV), 0)
        compute_half_chunk(lax.rem(my - 9 + N_DEV, N_DEV), HALF)
        compute_half_chunk(lax.rem(my + 6, N_DEV), HALF)

        for t in range(N_MIM):
            c_lor = lax.rem(my + 9 - t + N_DEV, N_DEV)
            c_hil = lax.rem(my - 9 + t + N_DEV, N_DEV)
            if t == 0:
                src_lor = acc_ref.at[lo_slice(c_lor)]
                src_hil = acc_ref.at[hi_slice(c_hil)]
            else:
                slot_lor_ref[t - 1, :, :] = (
                    slot_lor_ref[t - 1, :, :] + acc_ref[lo_slice(c_lor)])
                slot_hil_ref[t - 1, :, :] = (
                    slot_hil_ref[t - 1, :, :] + acc_ref[hi_slice(c_hil)])
                src_lor = slot_lor_ref.at[t - 1]
                src_hil = slot_hil_ref.at[t - 1]
            rdmas = [
                rs_copy(src_lor, slot_lor_ref, rs_lor_send, rs_lor_recv,
                        t, right),
                rs_copy(src_hil, slot_hil_ref, rs_hil_send, rs_hil_recv,
                        t, left),
            ]
            if t < N_MIM - 1:
                c_lol = lax.rem(my - 6 + t + N_DEV, N_DEV)
                c_hir = lax.rem(my + 6 - t + N_DEV, N_DEV)
                if t == 0:
                    src_lol = acc_ref.at[lo_slice(c_lol)]
                    src_hir = acc_ref.at[hi_slice(c_hir)]
                else:
                    slot_lol_ref[t - 1, :, :] = (
                        slot_lol_ref[t - 1, :, :] + acc_ref[lo_slice(c_lol)])
                    slot_hir_ref[t - 1, :, :] = (
                        slot_hir_ref[t - 1, :, :] + acc_ref[hi_slice(c_hir)])
                    src_lol = slot_lol_ref.at[t - 1]
                    src_hir = slot_hir_ref.at[t - 1]
                rdmas.append(rs_copy(src_lol, slot_lol_ref, rs_lol_send,
                                     rs_lol_recv, t, left))
                rdmas.append(rs_copy(src_hir, slot_hir_ref, rs_hir_send,
                                     rs_hir_recv, t, right))
            for rd in rdmas:
                rd.start()
            if t + 1 < N_MIM:
                compute_half_chunk(lax.rem(my + 9 - (t + 1) + N_DEV, N_DEV), 0)
                compute_half_chunk(lax.rem(my - 9 + (t + 1) + N_DEV, N_DEV),
                                   HALF)
            if t + 1 < N_MIM - 1:
                compute_half_chunk(lax.rem(my - 6 + (t + 1) + N_DEV, N_DEV), 0)
                compute_half_chunk(lax.rem(my + 6 - (t + 1) + N_DEV, N_DEV),
                                   HALF)
            if t == N_MIM - 2:
                compute_half_chunk(lax.rem(my + 1, N_DEV), 0)
                compute_half_chunk(lax.rem(my + N_DEV - 1, N_DEV), HALF)
            for rd in rdmas:
                rd.wait()

        r = lax.rem(my + 1, N_DEV)
        l = lax.rem(my + N_DEV - 1, N_DEV)
        agbuf_ref[lo_slice(r)] = (
            slot_lor_ref[N_MIM - 1, :, :] + slot_lol_ref[N_MIM - 2, :, :]
            + acc_ref[lo_slice(r)])
        agbuf_ref[hi_slice(l)] = (
            slot_hil_ref[N_MIM - 1, :, :] + slot_hir_ref[N_MIM - 2, :, :]
            + acc_ref[hi_slice(l)])

        N_MIM = N_DEV // 2

        def ag_copy(c, col_lo, sems_send, sems_recv, u, dev):
            return pltpu.make_async_remote_copy(
                src_ref=agbuf_ref.at[pl.ds(c * CHUNK, CHUNK), pl.ds(col_lo, HALF)],
                dst_ref=agbuf_ref.at[pl.ds(c * CHUNK, CHUNK), pl.ds(col_lo, HALF)],
                send_sem=sems_send.at[u],
                recv_sem=sems_recv.at[u],
                device_id=(dev,),
                device_id_type=pl.DeviceIdType.MESH,
            )

        def out_cvt(c, col_lo):
            sl = (pl.ds(c * CHUNK, CHUNK), pl.ds(col_lo, HALF))
            out_ref[sl] = agbuf_ref[sl].astype(jnp.float32)

        out_cvt(r, 0)
        out_cvt(l, HALF)
        for u in range(N_MIM):
            rdmas = [
                ag_copy(lax.rem(my - u + 1 + N_DEV, N_DEV), 0,
                        ag_rhr_send, ag_rhr_recv, u, right),
                ag_copy(lax.rem(my + u - 1 + N_DEV, N_DEV), HALF,
                        ag_lhl_send, ag_lhl_recv, u, left),
            ]
            if u < N_MIM - 1:
                rdmas.append(ag_copy(lax.rem(my + u + 1, N_DEV), 0,
                                     ag_rhl_send, ag_rhl_recv, u, left))
                rdmas.append(ag_copy(lax.rem(my - u - 1 + N_DEV, N_DEV), HALF,
                                     ag_lhr_send, ag_lhr_recv, u, right))
            for rd in rdmas:
                rd.start()
            if u >= 1:
                out_cvt(lax.rem(my - (u - 1) + N_DEV, N_DEV), 0)
                out_cvt(lax.rem(my + (u - 1), N_DEV), HALF)
                if u - 1 < N_MIM - 1:
                    out_cvt(lax.rem(my + u + 1, N_DEV), 0)
                    out_cvt(lax.rem(my - u - 1 + N_DEV, N_DEV), HALF)
            for rd in rdmas:
                rd.wait()
        out_cvt(lax.rem(my - (N_MIM - 1) + N_DEV, N_DEV), 0)
        out_cvt(lax.rem(my + N_MIM - 1, N_DEV), HALF)

    return pl.pallas_call(
        body,
        out_shape=jax.ShapeDtypeStruct((N_TOK, D_FF), jnp.float32),
        in_specs=[pl.BlockSpec(memory_space=pltpu.VMEM)] * 5,
        out_specs=pl.BlockSpec(memory_space=pltpu.VMEM),
        scratch_shapes=[
            pltpu.VMEM((N_TOK, D_FF), jnp.bfloat16),
            pltpu.VMEM((N_TOK, N_EXPERTS), jnp.float32),
            pltpu.VMEM((N_TOK, D_FF), jnp.bfloat16),
            pltpu.VMEM((8, CHUNK, HALF), jnp.bfloat16),
            pltpu.VMEM((7, CHUNK, HALF), jnp.bfloat16),
            pltpu.VMEM((8, CHUNK, HALF), jnp.bfloat16),
            pltpu.VMEM((7, CHUNK, HALF), jnp.bfloat16),
            pltpu.SemaphoreType.DMA((8,)),
            pltpu.SemaphoreType.DMA((8,)),
            pltpu.SemaphoreType.DMA((7,)),
            pltpu.SemaphoreType.DMA((7,)),
            pltpu.SemaphoreType.DMA((8,)),
            pltpu.SemaphoreType.DMA((8,)),
            pltpu.SemaphoreType.DMA((7,)),
            pltpu.SemaphoreType.DMA((7,)),
            pltpu.SemaphoreType.DMA((8,)),
            pltpu.SemaphoreType.DMA((8,)),
            pltpu.SemaphoreType.DMA((8,)),
            pltpu.SemaphoreType.DMA((8,)),
            pltpu.SemaphoreType.DMA((7,)),
            pltpu.SemaphoreType.DMA((7,)),
            pltpu.SemaphoreType.DMA((7,)),
            pltpu.SemaphoreType.DMA((7,)),
        ],
        compiler_params=pltpu.CompilerParams(
            collective_id=0,
            vmem_limit_bytes=100 * 1024 * 1024,
        ),
    )(x, router_W, route_idx, expert_W, shared_W)
